# baseline (speedup 1.0000x reference)
import functools

import jax
import jax.numpy as jnp
from jax.experimental import pallas as pl

B = 16384
N_IN = 256
SIZES = (128, 128, 128, 128, 64)
TOTALS = (256, 384, 512, 640, 768)
K = 16
BLK = 1024


def _densify_body(idx0, idx1, idx2, idx3, idx4, w0, w1, w2, w3, w4,
                  m0, m1, m2, m3, m4):
    idx_refs = (idx0, idx1, idx2, idx3, idx4)
    w_refs = (w0, w1, w2, w3, w4)
    m_refs = (m0, m1, m2, m3, m4)
    for li in range(5):
        sz = SIZES[li]
        rows = TOTALS[li]
        idx = idx_refs[li][...]
        idx = jnp.where(idx < N_IN, N_IN - 1 - idx, idx)
        w = w_refs[li][...]
        row_id = jax.lax.broadcasted_iota(jnp.int32, (rows, sz), 0)
        m = jnp.zeros((rows, sz), dtype=jnp.float32)
        for k in range(K):
            m = m + jnp.where(row_id == idx[k][None, :],
                              w[k][None, :], 0.0)
        m_refs[li][...] = m


def _forward_body(x_ref, m0, m1, m2, m3, m4, out_ref):
    dot = functools.partial(jnp.dot, preferred_element_type=jnp.float32,
                            precision=jax.lax.Precision.HIGHEST)
    x = x_ref[...]
    h0 = jnp.tanh(dot(x, m0[...]))
    h1 = jax.nn.relu(dot(x, m1[:256]) + dot(h0, m1[256:]))
    h2 = jax.nn.sigmoid(dot(x, m2[:256]) + dot(h0, m2[256:384])
                        + dot(h1, m2[384:]))
    h3 = jnp.tanh(dot(x, m3[:256]) + dot(h0, m3[256:384])
                  + dot(h1, m3[384:512]) + dot(h2, m3[512:]))
    out_ref[...] = (dot(x, m4[:256]) + dot(h0, m4[256:384])
                    + dot(h1, m4[384:512]) + dot(h2, m4[512:640])
                    + dot(h3, m4[640:]))


def kernel(x, idx0, idx1, idx2, idx3, idx4, w0, w1, w2, w3, w4):
    idxs = [a.T for a in (idx0, idx1, idx2, idx3, idx4)]
    ws = [a.T for a in (w0, w1, w2, w3, w4)]

    ms = pl.pallas_call(
        _densify_body,
        out_shape=[jax.ShapeDtypeStruct((TOTALS[li], SIZES[li]), jnp.float32)
                   for li in range(5)],
    )(*idxs, *ws)

    grid = (B // BLK,)
    out = pl.pallas_call(
        _forward_body,
        grid=grid,
        in_specs=[pl.BlockSpec((BLK, N_IN), lambda i: (i, 0))]
        + [pl.BlockSpec((TOTALS[li], SIZES[li]), lambda i: (0, 0))
           for li in range(5)],
        out_specs=pl.BlockSpec((BLK, SIZES[-1]), lambda i: (i, 0)),
        out_shape=jax.ShapeDtypeStruct((B, SIZES[-1]), jnp.float32),
    )(x, *ms)
    return out


# speedup vs baseline: 4.0805x; 4.0805x over previous
import functools

import jax
import jax.numpy as jnp
from jax.experimental import pallas as pl

B = 16384
N_IN = 256
SIZES = (128, 128, 128, 128, 64)
TOTALS = (256, 384, 512, 640, 768)
K = 16
BLK = 1024


def _densify_body(idx0, idx1, idx2, idx3, idx4, w0, w1, w2, w3, w4,
                  m0, m1, m2, m3, m4):
    idx_refs = (idx0, idx1, idx2, idx3, idx4)
    w_refs = (w0, w1, w2, w3, w4)
    m_refs = (m0, m1, m2, m3, m4)
    for li in range(5):
        sz = SIZES[li]
        rows = TOTALS[li]
        idx = idx_refs[li][...]
        idx = jnp.where(idx < N_IN, N_IN - 1 - idx, idx)
        w = w_refs[li][...]
        row_id = jax.lax.broadcasted_iota(jnp.int32, (rows, sz), 0)
        m = jnp.zeros((rows, sz), dtype=jnp.float32)
        for k in range(K):
            m = m + jnp.where(row_id == idx[k][None, :],
                              w[k][None, :], 0.0)
        m_refs[li][...] = m


def _forward_body(x_ref, m0, m1, m2, m3, m4, out_ref):
    dot = functools.partial(jnp.dot, preferred_element_type=jnp.float32)
    x = x_ref[...]
    h0 = jnp.tanh(dot(x, m0[...]))
    h1 = jax.nn.relu(dot(x, m1[:256]) + dot(h0, m1[256:]))
    h2 = jax.nn.sigmoid(dot(x, m2[:256]) + dot(h0, m2[256:384])
                        + dot(h1, m2[384:]))
    h3 = jnp.tanh(dot(x, m3[:256]) + dot(h0, m3[256:384])
                  + dot(h1, m3[384:512]) + dot(h2, m3[512:]))
    out_ref[...] = (dot(x, m4[:256]) + dot(h0, m4[256:384])
                    + dot(h1, m4[384:512]) + dot(h2, m4[512:640])
                    + dot(h3, m4[640:]))


def kernel(x, idx0, idx1, idx2, idx3, idx4, w0, w1, w2, w3, w4):
    idxs = [a.T for a in (idx0, idx1, idx2, idx3, idx4)]
    ws = [a.T for a in (w0, w1, w2, w3, w4)]

    ms = pl.pallas_call(
        _densify_body,
        out_shape=[jax.ShapeDtypeStruct((TOTALS[li], SIZES[li]), jnp.float32)
                   for li in range(5)],
    )(*idxs, *ws)

    grid = (B // BLK,)
    out = pl.pallas_call(
        _forward_body,
        grid=grid,
        in_specs=[pl.BlockSpec((BLK, N_IN), lambda i: (i, 0))]
        + [pl.BlockSpec((TOTALS[li], SIZES[li]), lambda i: (0, 0))
           for li in range(5)],
        out_specs=pl.BlockSpec((BLK, SIZES[-1]), lambda i: (i, 0)),
        out_shape=jax.ShapeDtypeStruct((B, SIZES[-1]), jnp.float32),
    )(x, *ms)
    return out


# speedup vs baseline: 4.2506x; 1.0417x over previous
import functools

import jax
import jax.numpy as jnp
from jax.experimental import pallas as pl
from jax.experimental.pallas import tpu as pltpu

B = 16384
N_IN = 256
SIZES = (128, 128, 128, 128, 64)
TOTALS = (256, 384, 512, 640, 768)
K = 16
BLK = 1024


def _body(idx0, idx1, idx2, idx3, idx4, w0, w1, w2, w3, w4, x_ref,
          out_ref, m0, m1, m2, m3, m4):
    idx_refs = (idx0, idx1, idx2, idx3, idx4)
    w_refs = (w0, w1, w2, w3, w4)
    m_refs = (m0, m1, m2, m3, m4)

    @pl.when(pl.program_id(0) == 0)
    def _densify():
        for li in range(5):
            sz = SIZES[li]
            rows = TOTALS[li]
            idx = idx_refs[li][...]
            idx = jnp.where(idx < N_IN, N_IN - 1 - idx, idx)
            w = w_refs[li][...]
            row_id = jax.lax.broadcasted_iota(jnp.int32, (rows, sz), 0)
            m = jnp.zeros((rows, sz), dtype=jnp.float32)
            for k in range(K):
                m = m + jnp.where(row_id == idx[k][None, :],
                                  w[k][None, :], 0.0)
            m_refs[li][...] = m

    dot = functools.partial(jnp.dot, preferred_element_type=jnp.float32)
    x = x_ref[...]
    h0 = jnp.tanh(dot(x, m0[...]))
    h1 = jax.nn.relu(dot(x, m1[:256]) + dot(h0, m1[256:]))
    h2 = jax.nn.sigmoid(dot(x, m2[:256]) + dot(h0, m2[256:384])
                        + dot(h1, m2[384:]))
    h3 = jnp.tanh(dot(x, m3[:256]) + dot(h0, m3[256:384])
                  + dot(h1, m3[384:512]) + dot(h2, m3[512:]))
    out_ref[...] = (dot(x, m4[:256]) + dot(h0, m4[256:384])
                    + dot(h1, m4[384:512]) + dot(h2, m4[512:640])
                    + dot(h3, m4[640:]))


def kernel(x, idx0, idx1, idx2, idx3, idx4, w0, w1, w2, w3, w4):
    idxs = [a.T for a in (idx0, idx1, idx2, idx3, idx4)]
    ws = [a.T for a in (w0, w1, w2, w3, w4)]

    grid = (B // BLK,)
    out = pl.pallas_call(
        _body,
        grid=grid,
        in_specs=[pl.BlockSpec((K, SIZES[li]), lambda i: (0, 0))
                  for li in range(5)] * 2
        + [pl.BlockSpec((BLK, N_IN), lambda i: (i, 0))],
        out_specs=pl.BlockSpec((BLK, SIZES[-1]), lambda i: (i, 0)),
        out_shape=jax.ShapeDtypeStruct((B, SIZES[-1]), jnp.float32),
        scratch_shapes=[pltpu.VMEM((TOTALS[li], SIZES[li]), jnp.float32)
                        for li in range(5)],
    )(*idxs, *ws, x)
    return out


# speedup vs baseline: 4.5227x; 1.0640x over previous
import functools

import jax
import jax.numpy as jnp
from jax.experimental import pallas as pl
from jax.experimental.pallas import tpu as pltpu

B = 16384
N_IN = 256
SIZES = (128, 128, 128, 128, 64)
TOTALS = (256, 384, 512, 640, 768)
K = 16
BLK = 2048


def _body(idx0, idx1, idx2, idx3, idx4, w0, w1, w2, w3, w4, x_ref,
          out_ref, m0, m1, m2, m3, m4):
    idx_refs = (idx0, idx1, idx2, idx3, idx4)
    w_refs = (w0, w1, w2, w3, w4)
    m_refs = (m0, m1, m2, m3, m4)

    @pl.when(pl.program_id(0) == 0)
    def _densify():
        for li in range(5):
            sz = SIZES[li]
            rows = TOTALS[li]
            idx = idx_refs[li][...]
            idx = jnp.where(idx < N_IN, N_IN - 1 - idx, idx)
            w = w_refs[li][...]
            row_id = jax.lax.broadcasted_iota(jnp.int32, (rows, sz), 0)
            m = jnp.zeros((rows, sz), dtype=jnp.float32)
            for k in range(K):
                m = m + jnp.where(row_id == idx[k][None, :],
                                  w[k][None, :], 0.0)
            m_refs[li][...] = m

    dot = functools.partial(jnp.dot, preferred_element_type=jnp.float32)
    x = x_ref[...]
    h0 = jnp.tanh(dot(x, m0[...]))
    h1 = jax.nn.relu(dot(x, m1[:256]) + dot(h0, m1[256:]))
    h2 = jax.nn.sigmoid(dot(x, m2[:256]) + dot(h0, m2[256:384])
                        + dot(h1, m2[384:]))
    h3 = jnp.tanh(dot(x, m3[:256]) + dot(h0, m3[256:384])
                  + dot(h1, m3[384:512]) + dot(h2, m3[512:]))
    out_ref[...] = (dot(x, m4[:256]) + dot(h0, m4[256:384])
                    + dot(h1, m4[384:512]) + dot(h2, m4[512:640])
                    + dot(h3, m4[640:]))


def kernel(x, idx0, idx1, idx2, idx3, idx4, w0, w1, w2, w3, w4):
    idxs = [a.T for a in (idx0, idx1, idx2, idx3, idx4)]
    ws = [a.T for a in (w0, w1, w2, w3, w4)]

    grid = (B // BLK,)
    out = pl.pallas_call(
        _body,
        grid=grid,
        in_specs=[pl.BlockSpec((K, SIZES[li]), lambda i: (0, 0))
                  for li in range(5)] * 2
        + [pl.BlockSpec((BLK, N_IN), lambda i: (i, 0))],
        out_specs=pl.BlockSpec((BLK, SIZES[-1]), lambda i: (i, 0)),
        out_shape=jax.ShapeDtypeStruct((B, SIZES[-1]), jnp.float32),
        scratch_shapes=[pltpu.VMEM((TOTALS[li], SIZES[li]), jnp.float32)
                        for li in range(5)],
    )(*idxs, *ws, x)
    return out


# speedup vs baseline: 4.5673x; 1.0099x over previous
import functools

import jax
import jax.numpy as jnp
from jax.experimental import pallas as pl
from jax.experimental.pallas import tpu as pltpu

B = 16384
N_IN = 256
SIZES = (128, 128, 128, 128, 64)
TOTALS = (256, 384, 512, 640, 768)
K = 16
BLK = 4096


def _body(idx0, idx1, idx2, idx3, idx4, w0, w1, w2, w3, w4, x_ref,
          out_ref, m0, m1, m2, m3, m4):
    idx_refs = (idx0, idx1, idx2, idx3, idx4)
    w_refs = (w0, w1, w2, w3, w4)
    m_refs = (m0, m1, m2, m3, m4)

    @pl.when(pl.program_id(0) == 0)
    def _densify():
        for li in range(5):
            sz = SIZES[li]
            rows = TOTALS[li]
            idx = idx_refs[li][...]
            idx = jnp.where(idx < N_IN, N_IN - 1 - idx, idx)
            w = w_refs[li][...]
            row_id = jax.lax.broadcasted_iota(jnp.int32, (rows, sz), 0)
            m = jnp.zeros((rows, sz), dtype=jnp.float32)
            for k in range(K):
                m = m + jnp.where(row_id == idx[k][None, :],
                                  w[k][None, :], 0.0)
            m_refs[li][...] = m

    dot = functools.partial(jnp.dot, preferred_element_type=jnp.float32)
    x = x_ref[...]
    h0 = jnp.tanh(dot(x, m0[...]))
    h1 = jax.nn.relu(dot(x, m1[:256]) + dot(h0, m1[256:]))
    h2 = jax.nn.sigmoid(dot(x, m2[:256]) + dot(h0, m2[256:384])
                        + dot(h1, m2[384:]))
    h3 = jnp.tanh(dot(x, m3[:256]) + dot(h0, m3[256:384])
                  + dot(h1, m3[384:512]) + dot(h2, m3[512:]))
    out_ref[...] = (dot(x, m4[:256]) + dot(h0, m4[256:384])
                    + dot(h1, m4[384:512]) + dot(h2, m4[512:640])
                    + dot(h3, m4[640:]))


def kernel(x, idx0, idx1, idx2, idx3, idx4, w0, w1, w2, w3, w4):
    idxs = [a.T for a in (idx0, idx1, idx2, idx3, idx4)]
    ws = [a.T for a in (w0, w1, w2, w3, w4)]

    grid = (B // BLK,)
    out = pl.pallas_call(
        _body,
        grid=grid,
        in_specs=[pl.BlockSpec((K, SIZES[li]), lambda i: (0, 0))
                  for li in range(5)] * 2
        + [pl.BlockSpec((BLK, N_IN), lambda i: (i, 0))],
        out_specs=pl.BlockSpec((BLK, SIZES[-1]), lambda i: (i, 0)),
        out_shape=jax.ShapeDtypeStruct((B, SIZES[-1]), jnp.float32),
        scratch_shapes=[pltpu.VMEM((TOTALS[li], SIZES[li]), jnp.float32)
                        for li in range(5)],
    )(*idxs, *ws, x)
    return out


# speedup vs baseline: 5.4447x; 1.1921x over previous
import functools

import jax
import jax.numpy as jnp
from jax.experimental import pallas as pl
from jax.experimental.pallas import tpu as pltpu

B = 16384
N_IN = 256
SIZES = (128, 128, 128, 128, 64)
TOTALS = (256, 384, 512, 640, 768)
K = 16
BLK = 4096
OFF = (0, 128, 256, 384, 512, 576)


def _body(idx0, idx1, idx2, idx3, idx4, w0, w1, w2, w3, w4, x_ref,
          out_ref, wx, w0a, w01, w2a, w23):
    idx_refs = (idx0, idx1, idx2, idx3, idx4)
    w_refs = (w0, w1, w2, w3, w4)

    @pl.when(pl.program_id(0) == 0)
    def _densify():
        for li in range(5):
            sz = SIZES[li]
            rows = TOTALS[li]
            idx = idx_refs[li][...]
            idx = jnp.where(idx < N_IN, N_IN - 1 - idx, idx)
            w = w_refs[li][...]
            row_id = jax.lax.broadcasted_iota(jnp.int32, (rows, sz), 0)
            m = jnp.zeros((rows, sz), dtype=jnp.float32)
            for k in range(K):
                m = m + jnp.where(row_id == idx[k][None, :],
                                  w[k][None, :], 0.0)
            c0, c1 = OFF[li], OFF[li] + sz
            wx[:, c0:c1] = m[:256]
            if li == 1:
                w0a[...] = m[256:384]
            if li >= 2:
                w01[0:128, c0 - 256:c1 - 256] = m[256:384]
                w01[128:256, c0 - 256:c1 - 256] = (
                    m[384:512] if rows > 384
                    else jnp.zeros((128, sz), jnp.float32))
            if li == 3:
                w2a[...] = m[512:640]
            if li == 4:
                w23[0:128, :] = m[512:640]
                w23[128:256, :] = m[640:768]

    dot = functools.partial(jnp.dot, preferred_element_type=jnp.float32)
    x = x_ref[...]
    X = dot(x, wx[...])
    h0 = jnp.tanh(X[:, 0:128])
    h1 = jax.nn.relu(X[:, 128:256] + dot(h0, w0a[...]))
    T = dot(jnp.concatenate([h0, h1], axis=1), w01[...])
    h2 = jax.nn.sigmoid(X[:, 256:384] + T[:, 0:128])
    h3 = jnp.tanh(X[:, 384:512] + T[:, 128:256] + dot(h2, w2a[...]))
    out_ref[...] = (X[:, 512:576] + T[:, 256:320]
                    + dot(jnp.concatenate([h2, h3], axis=1), w23[...]))


def kernel(x, idx0, idx1, idx2, idx3, idx4, w0, w1, w2, w3, w4):
    idxs = [a.T for a in (idx0, idx1, idx2, idx3, idx4)]
    ws = [a.T for a in (w0, w1, w2, w3, w4)]

    grid = (B // BLK,)
    out = pl.pallas_call(
        _body,
        grid=grid,
        in_specs=[pl.BlockSpec((K, SIZES[li]), lambda i: (0, 0))
                  for li in range(5)] * 2
        + [pl.BlockSpec((BLK, N_IN), lambda i: (i, 0))],
        out_specs=pl.BlockSpec((BLK, SIZES[-1]), lambda i: (i, 0)),
        out_shape=jax.ShapeDtypeStruct((B, SIZES[-1]), jnp.float32),
        scratch_shapes=[
            pltpu.VMEM((256, 576), jnp.float32),
            pltpu.VMEM((128, 128), jnp.float32),
            pltpu.VMEM((256, 320), jnp.float32),
            pltpu.VMEM((128, 128), jnp.float32),
            pltpu.VMEM((256, 64), jnp.float32),
        ],
    )(*idxs, *ws, x)
    return out
